# baseline (speedup 1.0000x reference)
import functools

import jax
import jax.numpy as jnp
from jax.experimental import pallas as pl
from jax.experimental.pallas import tpu as pltpu

_SPECS = [
    (64, 16, (3, 3, 3), (1, 1, 1)),
    (16, 16, (3, 3, 3), (1, 1, 1)),
    (16, 32, (3, 3, 3), (2, 2, 2)),
    (32, 32, (3, 3, 3), (1, 1, 1)),
    (32, 64, (3, 3, 3), (2, 2, 2)),
    (64, 64, (3, 3, 3), (1, 1, 1)),
    (64, 64, (3, 3, 3), (2, 2, 2)),
    (64, 64, (3, 3, 3), (1, 1, 1)),
    (64, 64, (3, 1, 1), (2, 1, 1)),
]
_B = 4
_G0 = (16, 48, 48)
_EPS = 1e-5

_BK_BY_S = {180000: 7200, 27040: 6760, 4704: 4704, 1024: 1024}
_RC_BY_S = {180000: 800, 27040: 520, 4704: 672, 1024: 512}


def _make_plans():
    plans = []
    g = _G0
    for (ci, co, k, s) in _SPECS:
        D, H, W = g
        Hp, Wp = H + 2, W + 2
        pads = tuple((kk - 1) // 2 for kk in k)
        groups = []
        for kd in range(k[0]):
            deltas = tuple((kh - pads[1]) * Wp + (kw - pads[2])
                           for kh in range(k[1]) for kw in range(k[2]))
            groups.append(((kd - pads[0]) * Hp * Wp, deltas))
        w1 = max(max(d, -d) for _, ds in groups for d in ds)
        m = max(max(gg, -gg) for gg, _ in groups) + w1
        S = _B * (D + 2) * Hp * Wp
        gout = tuple(gg // ss for gg, ss in zip(g, s))
        plans.append(dict(g=g, gout=gout, groups=tuple(groups), m=m, w1=w1,
                          S=S, Bk=_BK_BY_S[S], RC=_RC_BY_S[S], s=s,
                          strided=any(ss > 1 for ss in s)))
        g = gout
    return plans


_PLANS = _make_plans()


def _conv_body(xp, xc, xn, w, sc, sh,
               y, p1, p2, pc, xw, *, Bk, m, w1, groups, norm, Cin, Cout, RC):
    Ca = Cin + 2

    def nrm(part):
        if not norm:
            return part
        mcol = part[:, Cin:Cin + 1]
        keep = (jax.lax.broadcasted_iota(jnp.int32, (1, Ca), 1) == Cin + 1
                ).astype(jnp.float32)
        mul = mcol * (1.0 - keep) + keep
        return jnp.maximum(part * sc[...] + sh[...], 0.0) * mul

    xw[0:m, :] = nrm(xp[Bk - m:Bk, :])
    xw[m:m + Bk, :] = nrm(xc[...])
    xw[m + Bk:2 * m + Bk, :] = nrm(xn[0:m, :])

    def chunk(i, carry):
        p1c, p2c, pcc = carry
        base = i * RC
        acc = jnp.zeros((RC, Cout), jnp.float32)
        for gi, (g, deltas) in enumerate(groups):
            st = m + g - w1 + base
            ytv = jax.lax.dot_general(
                xw[pl.ds(st, RC + 2 * w1), :], w[gi],
                (((1,), (0,)), ((), ())), preferred_element_type=jnp.float32)
            for j, dl in enumerate(deltas):
                acc = acc + ytv[w1 + dl:w1 + dl + RC, j * Cout:(j + 1) * Cout]
        mcol = jnp.minimum(xw[pl.ds(m + base, RC), Cin:Cin + 1], 1.0)
        y[pl.ds(base, RC), :] = jnp.concatenate([acc, mcol, mcol], axis=1)
        sv = xw[pl.ds(m + base, RC), Cin + 1:Cin + 2]
        return (p1c + jax.lax.dot_general(
                    sv, acc, (((0,), (0,)), ((), ())),
                    precision=jax.lax.Precision.HIGHEST,
                    preferred_element_type=jnp.float32),
                p2c + jax.lax.dot_general(
                    sv, acc * acc, (((0,), (0,)), ((), ())),
                    precision=jax.lax.Precision.HIGHEST,
                    preferred_element_type=jnp.float32),
                pcc + jnp.sum(sv))

    init = (jnp.zeros((1, Cout), jnp.float32), jnp.zeros((1, Cout), jnp.float32),
            jnp.float32(0.0))
    p1v, p2v, pcv = jax.lax.fori_loop(0, Bk // RC, chunk, init)
    p1[...] = p1v.reshape(1, 1, Cout)
    p2[...] = p2v.reshape(1, 1, Cout)
    pc[...] = jnp.full((1, 1, 8), pcv, jnp.float32)


def _conv_layer(x_aug, w_g, scale, shift, plan, norm):
    S, Ca = x_aug.shape
    Cin = Ca - 2
    NG, _, NLC = w_g.shape
    groups = plan['groups']
    Cout = NLC // len(groups[0][1])
    Bk, m, w1 = plan['Bk'], plan['m'], plan['w1']
    Nb = S // Bk

    def ip(i):
        return (jnp.maximum(i - 1, 0), 0)

    def ic(i):
        return (i, 0)

    def inx(i):
        return (jnp.minimum(i + 1, Nb - 1), 0)

    body = functools.partial(_conv_body, Bk=Bk, m=m, w1=w1, groups=groups,
                             norm=norm, Cin=Cin, Cout=Cout, RC=plan['RC'])
    y, p1, p2, pc = pl.pallas_call(
        body,
        grid=(Nb,),
        in_specs=[
            pl.BlockSpec((Bk, Ca), ip),
            pl.BlockSpec((Bk, Ca), ic),
            pl.BlockSpec((Bk, Ca), inx),
            pl.BlockSpec((NG, Ca, NLC), lambda i: (0, 0, 0)),
            pl.BlockSpec((1, Ca), lambda i: (0, 0)),
            pl.BlockSpec((1, Ca), lambda i: (0, 0)),
        ],
        out_specs=[
            pl.BlockSpec((Bk, Cout + 2), ic),
            pl.BlockSpec((1, 1, Cout), lambda i: (i, 0, 0)),
            pl.BlockSpec((1, 1, Cout), lambda i: (i, 0, 0)),
            pl.BlockSpec((1, 1, 8), lambda i: (i, 0, 0)),
        ],
        out_shape=[
            jax.ShapeDtypeStruct((S, Cout + 2), jnp.float32),
            jax.ShapeDtypeStruct((Nb, 1, Cout), jnp.float32),
            jax.ShapeDtypeStruct((Nb, 1, Cout), jnp.float32),
            jax.ShapeDtypeStruct((Nb, 1, 8), jnp.float32),
        ],
        scratch_shapes=[
            pltpu.VMEM((Bk + 2 * m, Ca), jnp.float32),
        ],
    )(x_aug, x_aug, x_aug, w_g, scale, shift)
    return y, p1.sum(axis=(0, 1)), p2.sum(axis=(0, 1)), pc[:, 0, 0].sum()


def _norm_body(yr, scr, shr, orf, *, Cin):
    part = yr[...]
    mcol = part[:, Cin:Cin + 1]
    orf[...] = (jnp.maximum(part * scr[...] + shr[...], 0.0) * mcol)[:, :Cin]


def _pool_mask(mlane, g, s):
    D, H, W = g
    m5 = mlane.reshape(_B, D + 2, H + 2, W + 2)[:, 1:1 + D, 1:1 + H, 1:1 + W]
    m5 = m5.reshape(_B, D // s[0], s[0], H // s[1], s[1], W // s[2], s[2])
    return m5.max(axis=(2, 4, 6))


def _embed_sel(pooled, g, s):
    D, H, W = g
    z = jnp.zeros((_B, D + 2, H + 2, W + 2), jnp.float32)
    z = z.at[:, 1:1 + D:s[0], 1:1 + H:s[1], 1:1 + W:s[2]].set(pooled)
    return z.reshape(-1)


def _subsample(yflat, g, s):
    D, H, W = g
    C = yflat.shape[-1]
    y5 = yflat.reshape(_B, D + 2, H + 2, W + 2, C)
    return y5[:, 1:1 + D:s[0], 1:1 + H:s[1], 1:1 + W:s[2], :]


def _pad_flat(x5):
    C = x5.shape[-1]
    xp = jnp.pad(x5, ((0, 0), (1, 1), (1, 1), (1, 1), (0, 0)))
    return xp.reshape(-1, C)


def _aug_scale_shift(scale, shift):
    one = jnp.ones((1, 2), jnp.float32)
    zero = jnp.zeros((1, 2), jnp.float32)
    return (jnp.concatenate([scale, one], axis=1),
            jnp.concatenate([shift, zero], axis=1))


def kernel(voxel_features, coors, batch_size, input_shape, params):
    del batch_size, input_shape
    D0, H0, W0 = _G0
    Dp, Hp, Wp = D0 + 2, H0 + 2, W0 + 2
    b, d, h, w = coors[:, 0], coors[:, 1], coors[:, 2], coors[:, 3]
    r = ((b * Dp + (d + 1)) * Hp + (h + 1)) * Wp + (w + 1)
    S0 = _B * Dp * Hp * Wp
    feat = jnp.zeros((S0, 64), jnp.float32).at[r].add(voxel_features)
    mask_lane = jnp.zeros((S0,), jnp.float32).at[r].set(1.0)
    x_aug = jnp.concatenate(
        [feat, mask_lane[:, None], mask_lane[:, None]], axis=1)

    scale = jnp.ones((1, 66), jnp.float32)
    shift = jnp.zeros((1, 66), jnp.float32)

    for li, (spec, plan) in enumerate(zip(_SPECS, _PLANS)):
        ci, co, k, s = spec
        nkd = k[0]
        nloc = k[1] * k[2]
        w_g = jnp.transpose(params[li]['W'], (2, 1, 3, 4, 0)).reshape(
            nkd, ci, nloc * co)
        w_g = jnp.pad(w_g, ((0, 0), (0, 2), (0, 0)))
        if plan['strided']:
            pooled = _pool_mask(mask_lane, plan['g'], s)
            sel_lane = _embed_sel(pooled, plan['g'], s)
            x_aug = x_aug.at[:, ci + 1].set(sel_lane)
        else:
            pooled = None
        y_aug, p1, p2, pcnt = _conv_layer(x_aug, w_g, scale, shift,
                                          plan, norm=(li > 0))
        cnt = jnp.maximum(pcnt, 1.0)
        mu = p1 / cnt
        var = p2 / cnt - mu * mu
        inv = jax.lax.rsqrt(var + _EPS)
        scale_c = (params[li]['gamma'] * inv).reshape(1, co)
        shift_c = (params[li]['beta'] - mu * params[li]['gamma'] * inv).reshape(1, co)
        scale, shift = _aug_scale_shift(scale_c, shift_c)
        if plan['strided']:
            x_core = _pad_flat(_subsample(y_aug[:, :co], plan['g'], s))
            mask_lane = _pad_flat(pooled[..., None])[:, 0]
            x_aug = jnp.concatenate(
                [x_core, mask_lane[:, None], mask_lane[:, None]], axis=1)
        else:
            x_aug = y_aug

    Sf, Ca = x_aug.shape
    Cf = Ca - 2
    out = pl.pallas_call(
        functools.partial(_norm_body, Cin=Cf),
        out_shape=jax.ShapeDtypeStruct((Sf, Cf), jnp.float32),
    )(x_aug, scale, shift)

    Df, Hf, Wf = _PLANS[-1]['gout']
    o5 = out.reshape(_B, Df + 2, Hf + 2, Wf + 2, Cf)
    o5 = o5[:, 1:1 + Df, 1:1 + Hf, 1:1 + Wf, :]
    return jnp.transpose(o5, (0, 4, 1, 2, 3))


# speedup vs baseline: 1.0928x; 1.0928x over previous
import functools

import jax
import jax.numpy as jnp
from jax.experimental import pallas as pl
from jax.experimental.pallas import tpu as pltpu

_SPECS = [
    (64, 16, (3, 3, 3), (1, 1, 1)),
    (16, 16, (3, 3, 3), (1, 1, 1)),
    (16, 32, (3, 3, 3), (2, 2, 2)),
    (32, 32, (3, 3, 3), (1, 1, 1)),
    (32, 64, (3, 3, 3), (2, 2, 2)),
    (64, 64, (3, 3, 3), (1, 1, 1)),
    (64, 64, (3, 3, 3), (2, 2, 2)),
    (64, 64, (3, 3, 3), (1, 1, 1)),
    (64, 64, (3, 1, 1), (2, 1, 1)),
]
_B = 4
_G0 = (16, 48, 48)
_EPS = 1e-5

_BLK_BY_R = {201600: (4032, 576), 33280: (4160, 520),
             5376: (5376, 672), 1024: (1024, 512)}


def _wl(W):
    return (W + 2 + 7) // 8 * 8


def _make_plans():
    plans = []
    g = _G0
    for (ci, co, k, s) in _SPECS:
        D, H, W = g
        Dp, Hp, WL = D + 2, H + 2, _wl(W)
        kw3 = k[2] == 3
        taps = []
        for kd in range(k[0]):
            for kh in range(k[1]):
                taps.append((kd - (k[0] - 1) // 2) * Hp * WL
                            + (kh - (k[1] - 1) // 2) * WL)
        m = max(max(t, -t) for t in taps)
        if m % 8:
            m += 8 - m % 8
        if kw3:
            m += 8
        R = _B * Dp * Hp * WL
        Bk, RC = _BLK_BY_R[R]
        gout = tuple(gg // ss for gg, ss in zip(g, s))
        plans.append(dict(g=g, gout=gout, taps=tuple(taps), m=m, R=R, Bk=Bk,
                          RC=RC, s=s, kw3=kw3,
                          strided=any(ss > 1 for ss in s)))
        g = gout
    return plans


_PLANS = _make_plans()


def _conv_body(xp, xc, xn, w, sc, sh, y, p1, p2, pc, *scratch,
               Bk, m, taps, norm, Cin, Cout, RC, kw3):
    Ca = Cin + 2
    xw = scratch[0]

    def nrm(part):
        if not norm:
            return part
        mcol = part[:, Cin:Cin + 1]
        keep = (jax.lax.broadcasted_iota(jnp.int32, (1, Ca), 1) == Cin + 1
                ).astype(jnp.float32)
        mul = mcol * (1.0 - keep) + keep
        return jnp.maximum(part * sc[...] + sh[...], 0.0) * mul

    xw[0:m, :] = nrm(xp[Bk - m:Bk, :])
    xw[m:m + Bk, :] = nrm(xc[...])
    xw[m + Bk:2 * m + Bk, :] = nrm(xn[0:m, :])

    L = Bk + 2 * m
    if kw3:
        x3 = scratch[1]
        x3[0:8, :] = jnp.zeros((8, x3.shape[1]), jnp.float32)
        x3[L - 8:L, :] = jnp.zeros((8, x3.shape[1]), jnp.float32)
        npieces = 8
        pb = (L - 16) // npieces + 1
        lo = 8
        while lo < L - 8:
            hi = min(lo + pb, L - 8)
            x3[lo:hi, :] = jnp.concatenate(
                [xw[lo - 1:hi - 1, :], xw[lo:hi, :], xw[lo + 1:hi + 1, :]],
                axis=1)
            lo = hi
        src = x3
    else:
        src = xw

    def chunk(i, carry):
        p1c, p2c, pcc = carry
        base = i * RC
        acc = jnp.zeros((RC, Cout), jnp.float32)
        for t, shift in enumerate(taps):
            st = pl.multiple_of(m + shift + base, 8)
            acc = acc + jax.lax.dot_general(
                src[pl.ds(st, RC), :], w[t],
                (((1,), (0,)), ((), ())), preferred_element_type=jnp.float32)
        cbase = pl.multiple_of(m + base, 8)
        mcol = jnp.minimum(xw[pl.ds(cbase, RC), Cin:Cin + 1], 1.0)
        y[pl.ds(base, RC), :] = jnp.concatenate([acc, mcol, mcol], axis=1)
        sv = xw[pl.ds(cbase, RC), Cin + 1:Cin + 2]
        return (p1c + jax.lax.dot_general(
                    sv, acc, (((0,), (0,)), ((), ())),
                    precision=jax.lax.Precision.HIGHEST,
                    preferred_element_type=jnp.float32),
                p2c + jax.lax.dot_general(
                    sv, acc * acc, (((0,), (0,)), ((), ())),
                    precision=jax.lax.Precision.HIGHEST,
                    preferred_element_type=jnp.float32),
                pcc + jnp.sum(sv))

    init = (jnp.zeros((1, Cout), jnp.float32), jnp.zeros((1, Cout), jnp.float32),
            jnp.float32(0.0))
    p1v, p2v, pcv = jax.lax.fori_loop(0, Bk // RC, chunk, init)
    p1[...] = p1v.reshape(1, 1, Cout)
    p2[...] = p2v.reshape(1, 1, Cout)
    pc[...] = jnp.full((1, 1, 8), pcv, jnp.float32)


def _conv_layer(x_aug, w_g, scale, shift, plan, norm):
    R, Ca = x_aug.shape
    Cin = Ca - 2
    NT, KK, Cout = w_g.shape
    taps = plan['taps']
    Bk, m, kw3 = plan['Bk'], plan['m'], plan['kw3']
    Nb = R // Bk

    def ip(i):
        return (jnp.maximum(i - 1, 0), 0)

    def ic(i):
        return (i, 0)

    def inx(i):
        return (jnp.minimum(i + 1, Nb - 1), 0)

    scratch = [pltpu.VMEM((Bk + 2 * m, Ca), jnp.float32)]
    if kw3:
        scratch.append(pltpu.VMEM((Bk + 2 * m, 3 * Ca), jnp.float32))
    body = functools.partial(_conv_body, Bk=Bk, m=m, taps=taps,
                             norm=norm, Cin=Cin, Cout=Cout, RC=plan['RC'],
                             kw3=kw3)
    y, p1, p2, pc = pl.pallas_call(
        body,
        grid=(Nb,),
        in_specs=[
            pl.BlockSpec((Bk, Ca), ip),
            pl.BlockSpec((Bk, Ca), ic),
            pl.BlockSpec((Bk, Ca), inx),
            pl.BlockSpec((NT, KK, Cout), lambda i: (0, 0, 0)),
            pl.BlockSpec((1, Ca), lambda i: (0, 0)),
            pl.BlockSpec((1, Ca), lambda i: (0, 0)),
        ],
        out_specs=[
            pl.BlockSpec((Bk, Cout + 2), ic),
            pl.BlockSpec((1, 1, Cout), lambda i: (i, 0, 0)),
            pl.BlockSpec((1, 1, Cout), lambda i: (i, 0, 0)),
            pl.BlockSpec((1, 1, 8), lambda i: (i, 0, 0)),
        ],
        out_shape=[
            jax.ShapeDtypeStruct((R, Cout + 2), jnp.float32),
            jax.ShapeDtypeStruct((Nb, 1, Cout), jnp.float32),
            jax.ShapeDtypeStruct((Nb, 1, Cout), jnp.float32),
            jax.ShapeDtypeStruct((Nb, 1, 8), jnp.float32),
        ],
        scratch_shapes=scratch,
    )(x_aug, x_aug, x_aug, w_g, scale, shift)
    return y, p1.sum(axis=(0, 1)), p2.sum(axis=(0, 1)), pc[:, 0, 0].sum()


def _norm_body(yr, scr, shr, orf, *, Cin):
    part = yr[...]
    mcol = part[:, Cin:Cin + 1]
    orf[...] = (jnp.maximum(part * scr[...] + shr[...], 0.0) * mcol)[:, :Cin]


def _pool_mask(mlane, g, s):
    D, H, W = g
    m5 = mlane.reshape(_B, D + 2, H + 2, _wl(W))[:, 1:1 + D, 1:1 + H, 1:1 + W]
    m5 = m5.reshape(_B, D // s[0], s[0], H // s[1], s[1], W // s[2], s[2])
    return m5.max(axis=(2, 4, 6))


def _embed_sel(pooled, g, s):
    D, H, W = g
    z = jnp.zeros((_B, D + 2, H + 2, _wl(W)), jnp.float32)
    z = z.at[:, 1:1 + D:s[0], 1:1 + H:s[1], 1:1 + W:s[2]].set(pooled)
    return z.reshape(-1)


def _subsample(yflat, g, s):
    D, H, W = g
    C = yflat.shape[-1]
    y5 = yflat.reshape(_B, D + 2, H + 2, _wl(W), C)
    return y5[:, 1:1 + D:s[0], 1:1 + H:s[1], 1:1 + W:s[2], :]


def _pad_flat(x5, gnew):
    D, H, W = gnew
    C = x5.shape[-1]
    xp = jnp.pad(x5, ((0, 0), (1, 1), (1, 1), (1, _wl(W) - W - 1), (0, 0)))
    return xp.reshape(-1, C)


def _aug_scale_shift(scale, shift):
    one = jnp.ones((1, 2), jnp.float32)
    zero = jnp.zeros((1, 2), jnp.float32)
    return (jnp.concatenate([scale, one], axis=1),
            jnp.concatenate([shift, zero], axis=1))


def kernel(voxel_features, coors, batch_size, input_shape, params):
    del batch_size, input_shape
    D0, H0, W0 = _G0
    Dp, Hp, WL = D0 + 2, H0 + 2, _wl(W0)
    b, d, h, w = coors[:, 0], coors[:, 1], coors[:, 2], coors[:, 3]
    r = ((b * Dp + (d + 1)) * Hp + (h + 1)) * WL + (w + 1)
    R0 = _B * Dp * Hp * WL
    feat = jnp.zeros((R0, 64), jnp.float32).at[r].add(voxel_features)
    mask_lane = jnp.zeros((R0,), jnp.float32).at[r].set(1.0)
    x_aug = jnp.concatenate(
        [feat, mask_lane[:, None], mask_lane[:, None]], axis=1)

    scale = jnp.ones((1, 66), jnp.float32)
    shift = jnp.zeros((1, 66), jnp.float32)

    for li, (spec, plan) in enumerate(zip(_SPECS, _PLANS)):
        ci, co, k, s = spec
        ca = ci + 2
        if plan['kw3']:
            wt = jnp.transpose(params[li]['W'], (2, 3, 4, 1, 0))
            wt = wt.reshape(k[0] * k[1], k[2], ci, co)
            wt = jnp.pad(wt, ((0, 0), (0, 0), (0, 2), (0, 0)))
            w_g = wt.reshape(k[0] * k[1], k[2] * ca, co)
        else:
            wt = jnp.transpose(params[li]['W'], (2, 3, 4, 1, 0))
            wt = wt.reshape(k[0] * k[1], ci, co)
            w_g = jnp.pad(wt, ((0, 0), (0, 2), (0, 0)))
        if plan['strided']:
            pooled = _pool_mask(mask_lane, plan['g'], s)
            sel_lane = _embed_sel(pooled, plan['g'], s)
            x_aug = x_aug.at[:, ci + 1].set(sel_lane)
        else:
            pooled = None
        y_aug, p1, p2, pcnt = _conv_layer(x_aug, w_g, scale, shift,
                                          plan, norm=(li > 0))
        cnt = jnp.maximum(pcnt, 1.0)
        mu = p1 / cnt
        var = p2 / cnt - mu * mu
        inv = jax.lax.rsqrt(var + _EPS)
        scale_c = (params[li]['gamma'] * inv).reshape(1, co)
        shift_c = (params[li]['beta'] - mu * params[li]['gamma'] * inv).reshape(1, co)
        scale, shift = _aug_scale_shift(scale_c, shift_c)
        if plan['strided']:
            x_core = _pad_flat(_subsample(y_aug[:, :co], plan['g'], s),
                               plan['gout'])
            mask_lane = _pad_flat(pooled[..., None], plan['gout'])[:, 0]
            x_aug = jnp.concatenate(
                [x_core, mask_lane[:, None], mask_lane[:, None]], axis=1)
        else:
            x_aug = y_aug

    Sf, Ca = x_aug.shape
    Cf = Ca - 2
    out = pl.pallas_call(
        functools.partial(_norm_body, Cin=Cf),
        out_shape=jax.ShapeDtypeStruct((Sf, Cf), jnp.float32),
    )(x_aug, scale, shift)

    Df, Hf, Wf = _PLANS[-1]['gout']
    o5 = out.reshape(_B, Df + 2, Hf + 2, _wl(Wf), Cf)
    o5 = o5[:, 1:1 + Df, 1:1 + Hf, 1:1 + Wf, :]
    return jnp.transpose(o5, (0, 4, 1, 2, 3))


# speedup vs baseline: 1.5441x; 1.4130x over previous
import functools

import jax
import jax.numpy as jnp
from jax.experimental import pallas as pl
from jax.experimental.pallas import tpu as pltpu

_SPECS = [
    (64, 16, (3, 3, 3), (1, 1, 1)),
    (16, 16, (3, 3, 3), (1, 1, 1)),
    (16, 32, (3, 3, 3), (2, 2, 2)),
    (32, 32, (3, 3, 3), (1, 1, 1)),
    (32, 64, (3, 3, 3), (2, 2, 2)),
    (64, 64, (3, 3, 3), (1, 1, 1)),
    (64, 64, (3, 3, 3), (2, 2, 2)),
    (64, 64, (3, 3, 3), (1, 1, 1)),
    (64, 64, (3, 1, 1), (2, 1, 1)),
]
_B = 4
_G0 = (16, 48, 48)
_EPS = 1e-5

_BK_BY_S = {180000: 3600, 27040: 1352, 4704: 1176, 1024: 1024}


def _make_plans():
    plans = []
    g = _G0
    for (ci, co, k, s) in _SPECS:
        D, H, W = g
        Hp, Wp = H + 2, W + 2
        pads = tuple((kk - 1) // 2 for kk in k)
        offs = [
            (kd - pads[0]) * Hp * Wp + (kh - pads[1]) * Wp + (kw - pads[2])
            for kd in range(k[0])
            for kh in range(k[1])
            for kw in range(k[2])
        ]
        m = max(max(offs), -min(offs))
        S = _B * (D + 2) * Hp * Wp
        Bk = _BK_BY_S[S]
        gout = tuple(gg // ss for gg, ss in zip(g, s))
        plans.append(dict(g=g, gout=gout, offs=tuple(offs), m=m, S=S, Bk=Bk,
                          s=s, strided=any(ss > 1 for ss in s)))
        g = gout
    return plans


_PLANS = _make_plans()


def _conv_body(xp, xc, xn, mp, mc, mn, sel, w, sc, sh,
               y, p1, p2, pc, *, Bk, m, offs, norm, Cout):
    window = jnp.concatenate([xp[...], xc[...], xn[...]], axis=0)
    need = window[Bk - m:2 * Bk + m, :]
    if norm:
        mwin = jnp.concatenate([mp[...], mc[...], mn[...]], axis=0)
        mwin = mwin[Bk - m:2 * Bk + m, :]
        need = jnp.maximum(need * sc[...] + sh[...], 0.0) * mwin
    acc = jnp.zeros((Bk, Cout), jnp.float32)
    for t, off in enumerate(offs):
        sl = need[m + off:m + off + Bk, :]
        acc = acc + jax.lax.dot_general(
            sl, w[t], (((1,), (0,)), ((), ())),
            preferred_element_type=jnp.float32)
    y[...] = acc
    s = sel[...]
    p1[...] = jnp.sum(acc * s, axis=0).reshape(1, 1, Cout)
    p2[...] = jnp.sum(acc * acc * s, axis=0).reshape(1, 1, Cout)
    pc[...] = jnp.full((1, 1, 8), jnp.sum(s), jnp.float32)


def _conv_layer(x, mask, sel, w_taps, scale, shift, plan, norm):
    S, Cin = x.shape
    T, _, Cout = w_taps.shape
    Bk, m, offs = plan['Bk'], plan['m'], plan['offs']
    Nb = S // Bk

    def ip(i):
        return (jnp.maximum(i - 1, 0), 0)

    def ic(i):
        return (i, 0)

    def inx(i):
        return (jnp.minimum(i + 1, Nb - 1), 0)

    body = functools.partial(_conv_body, Bk=Bk, m=m, offs=offs, norm=norm,
                             Cout=Cout)
    y, p1, p2, pc = pl.pallas_call(
        body,
        grid=(Nb,),
        in_specs=[
            pl.BlockSpec((Bk, Cin), ip),
            pl.BlockSpec((Bk, Cin), ic),
            pl.BlockSpec((Bk, Cin), inx),
            pl.BlockSpec((Bk, 1), ip),
            pl.BlockSpec((Bk, 1), ic),
            pl.BlockSpec((Bk, 1), inx),
            pl.BlockSpec((Bk, 1), ic),
            pl.BlockSpec((T, Cin, Cout), lambda i: (0, 0, 0)),
            pl.BlockSpec((1, Cin), lambda i: (0, 0)),
            pl.BlockSpec((1, Cin), lambda i: (0, 0)),
        ],
        out_specs=[
            pl.BlockSpec((Bk, Cout), ic),
            pl.BlockSpec((1, 1, Cout), lambda i: (i, 0, 0)),
            pl.BlockSpec((1, 1, Cout), lambda i: (i, 0, 0)),
            pl.BlockSpec((1, 1, 8), lambda i: (i, 0, 0)),
        ],
        out_shape=[
            jax.ShapeDtypeStruct((S, Cout), jnp.float32),
            jax.ShapeDtypeStruct((Nb, 1, Cout), jnp.float32),
            jax.ShapeDtypeStruct((Nb, 1, Cout), jnp.float32),
            jax.ShapeDtypeStruct((Nb, 1, 8), jnp.float32),
        ],
    )(x, x, x, mask, mask, mask, sel, w_taps, scale, shift)
    return y, p1.sum(axis=(0, 1)), p2.sum(axis=(0, 1)), pc[:, 0, 0].sum()


def _norm_body(yr, mr, scr, shr, orf):
    orf[...] = jnp.maximum(yr[...] * scr[...] + shr[...], 0.0) * mr[...]


def _pool_mask(mflat, g, s):
    D, H, W = g
    m5 = mflat.reshape(_B, D + 2, H + 2, W + 2)[:, 1:1 + D, 1:1 + H, 1:1 + W]
    m5 = m5.reshape(_B, D // s[0], s[0], H // s[1], s[1], W // s[2], s[2])
    return m5.max(axis=(2, 4, 6))


def _embed_sel(pooled, g, s):
    D, H, W = g
    z = jnp.zeros((_B, D + 2, H + 2, W + 2), jnp.float32)
    z = z.at[:, 1:1 + D:s[0], 1:1 + H:s[1], 1:1 + W:s[2]].set(pooled)
    return z.reshape(-1, 1)


def _subsample(yflat, g, s):
    D, H, W = g
    C = yflat.shape[-1]
    y5 = yflat.reshape(_B, D + 2, H + 2, W + 2, C)
    return y5[:, 1:1 + D:s[0], 1:1 + H:s[1], 1:1 + W:s[2], :]


def _pad_flat(x5):
    C = x5.shape[-1]
    xp = jnp.pad(x5, ((0, 0), (1, 1), (1, 1), (1, 1), (0, 0)))
    return xp.reshape(-1, C)


def kernel(voxel_features, coors, batch_size, input_shape, params):
    del batch_size, input_shape
    D0, H0, W0 = _G0
    Dp, Hp, Wp = D0 + 2, H0 + 2, W0 + 2
    b, d, h, w = coors[:, 0], coors[:, 1], coors[:, 2], coors[:, 3]
    r = ((b * Dp + (d + 1)) * Hp + (h + 1)) * Wp + (w + 1)
    S0 = _B * Dp * Hp * Wp
    x = jnp.zeros((S0, 64), jnp.float32).at[r].add(voxel_features)
    mask = jnp.zeros((S0, 1), jnp.float32).at[r].set(1.0)

    Cin0 = 64
    scale = jnp.ones((1, Cin0), jnp.float32)
    shift = jnp.zeros((1, Cin0), jnp.float32)

    for li, (spec, plan) in enumerate(zip(_SPECS, _PLANS)):
        ci, co, k, s = spec
        T = len(plan['offs'])
        w_taps = jnp.transpose(params[li]['W'], (2, 3, 4, 1, 0)).reshape(T, ci, co)
        if plan['strided']:
            pooled = _pool_mask(mask, plan['g'], s)
            sel = _embed_sel(pooled, plan['g'], s)
        else:
            pooled = None
            sel = mask
        y, p1, p2, pcnt = _conv_layer(x, mask, sel, w_taps, scale, shift,
                                      plan, norm=(li > 0))
        cnt = jnp.maximum(pcnt, 1.0)
        mu = p1 / cnt
        var = p2 / cnt - mu * mu
        inv = jax.lax.rsqrt(var + _EPS)
        scale = (params[li]['gamma'] * inv).reshape(1, co)
        shift = (params[li]['beta'] - mu * params[li]['gamma'] * inv).reshape(1, co)
        if plan['strided']:
            x = _pad_flat(_subsample(y, plan['g'], s))
            mask = _pad_flat(pooled[..., None])
        else:
            x = y

    Sf, Cf = x.shape
    out = pl.pallas_call(
        _norm_body,
        out_shape=jax.ShapeDtypeStruct((Sf, Cf), jnp.float32),
    )(x, mask, scale, shift)

    Df, Hf, Wf = _PLANS[-1]['gout']
    o5 = out.reshape(_B, Df + 2, Hf + 2, Wf + 2, Cf)
    o5 = o5[:, 1:1 + Df, 1:1 + Hf, 1:1 + Wf, :]
    return jnp.transpose(o5, (0, 4, 1, 2, 3))
